# baseline (speedup 1.0000x reference)
import jax
import jax.numpy as jnp
from jax import lax
from jax.experimental import pallas as pl
from jax.experimental.pallas import tpu as pltpu
from jax.experimental.pallas import tpu_sc as plsc

VOCAB = 1000000
D = 32
BATCH = 16384
HIST = 50
NUM_CORES = 2
NUM_SUBCORES = 16
NW = NUM_CORES * NUM_SUBCORES
B_PER_W = BATCH // NW
B_TOTAL = BATCH * HIST


def _detile_kernel(idx_hbm, out_hbm, idx_v):
    wid = lax.axis_index("s") * NUM_CORES + lax.axis_index("c")
    b0 = wid * B_PER_W
    for q in (0, 8, 16, 24, 32, 40):
        pltpu.sync_copy(idx_hbm.at[pl.ds(q, 8), pl.ds(b0, B_PER_W)],
                        idx_v.at[pl.ds(q, 8)])
    pltpu.sync_copy(idx_hbm.at[pl.ds(48, 2), pl.ds(b0, B_PER_W)],
                    idx_v.at[pl.ds(48, 2)])
    for h in range(HIST):
        pltpu.sync_copy(idx_v.at[h],
                        out_hbm.at[pl.ds(h * BATCH + b0, B_PER_W)])


def _gather_kernel(table_hbm, idx_hbm, out_hbm,
                   idx_v, rows0, rows1, gs0, gs1, os0, os1):
    wid = lax.axis_index("s") * NUM_CORES + lax.axis_index("c")
    b0 = wid * B_PER_W

    rows = (rows0, rows1)
    gsem = (gs0, gs1)
    osem = (os0, os1)
    g = [None, None]
    o = [None, None]

    for h in range(HIST):
        pltpu.sync_copy(idx_hbm.at[pl.ds(h * BATCH + b0, B_PER_W)],
                        idx_v.at[h])

    g[0] = pltpu.async_copy(table_hbm.at[idx_v.at[0]], rows[0], gsem[0])
    for h in range(HIST):
        b = h & 1
        nb = b ^ 1
        if h + 1 < HIST:
            if o[nb] is not None:
                o[nb].wait()
            g[nb] = pltpu.async_copy(
                table_hbm.at[idx_v.at[h + 1]], rows[nb], gsem[nb])
        g[b].wait()
        o[b] = pltpu.async_copy(
            rows[b], out_hbm.at[pl.ds(h * BATCH + b0, B_PER_W)], osem[b])
    o[0].wait()
    o[1].wait()


def kernel(input, embedding_matrix):
    idx_t = input.T
    mesh = plsc.VectorSubcoreMesh(core_axis_name="c", subcore_axis_name="s")
    idx_flat = pl.kernel(
        _detile_kernel,
        out_type=jax.ShapeDtypeStruct((B_TOTAL,), jnp.int32),
        mesh=mesh,
        scratch_types=[
            pltpu.VMEM((56, B_PER_W), jnp.int32),
        ],
        compiler_params=pltpu.CompilerParams(use_tc_tiling_on_sc=True),
    )(idx_t)

    out = pl.kernel(
        _gather_kernel,
        out_type=jax.ShapeDtypeStruct((B_TOTAL, D), jnp.float32),
        mesh=mesh,
        scratch_types=[
            pltpu.VMEM((HIST, B_PER_W), jnp.int32),
            pltpu.VMEM((B_PER_W, D), jnp.float32),
            pltpu.VMEM((B_PER_W, D), jnp.float32),
            pltpu.SemaphoreType.DMA,
            pltpu.SemaphoreType.DMA,
            pltpu.SemaphoreType.DMA,
            pltpu.SemaphoreType.DMA,
        ],
        compiler_params=pltpu.CompilerParams(use_tc_tiling_on_sc=False),
    )(embedding_matrix, idx_flat)
    return out.reshape(HIST, BATCH, D).transpose(1, 0, 2)


# speedup vs baseline: 1.0227x; 1.0227x over previous
import jax
import jax.numpy as jnp
from jax import lax
from jax.experimental import pallas as pl
from jax.experimental.pallas import tpu as pltpu
from jax.experimental.pallas import tpu_sc as plsc

VOCAB = 1000000
D = 32
BATCH = 16384
HIST = 50
NUM_CORES = 2
NUM_SUBCORES = 16
NW = NUM_CORES * NUM_SUBCORES
B_PER_W = BATCH // NW


def _gather_kernel(table_hbm, idx_hbm, out_hbm,
                   idx_v, rows0, rows1, gs0, gs1, os0, os1):
    wid = lax.axis_index("s") * NUM_CORES + lax.axis_index("c")
    b0 = wid * B_PER_W
    pltpu.sync_copy(idx_hbm.at[:, pl.ds(b0, B_PER_W)], idx_v)

    rows = (rows0, rows1)
    gsem = (gs0, gs1)
    osem = (os0, os1)
    g = [None, None]
    o = [None, None]

    g[0] = pltpu.async_copy(table_hbm.at[idx_v.at[0]], rows[0], gsem[0])
    for h in range(HIST):
        b = h & 1
        nb = b ^ 1
        if h + 1 < HIST:
            if o[nb] is not None:
                o[nb].wait()
            g[nb] = pltpu.async_copy(
                table_hbm.at[idx_v.at[h + 1]], rows[nb], gsem[nb])
        g[b].wait()
        o[b] = pltpu.async_copy(
            rows[b], out_hbm.at[h, pl.ds(b0, B_PER_W)], osem[b])
    o[0].wait()
    o[1].wait()


def kernel(input, embedding_matrix):
    idx_t = input.T
    mesh = plsc.VectorSubcoreMesh(core_axis_name="c", subcore_axis_name="s")
    out = pl.kernel(
        _gather_kernel,
        out_type=jax.ShapeDtypeStruct((HIST, BATCH, D), jnp.float32),
        mesh=mesh,
        scratch_types=[
            pltpu.VMEM((HIST, B_PER_W), jnp.int32),
            pltpu.VMEM((B_PER_W, D), jnp.float32),
            pltpu.VMEM((B_PER_W, D), jnp.float32),
            pltpu.SemaphoreType.DMA,
            pltpu.SemaphoreType.DMA,
            pltpu.SemaphoreType.DMA,
            pltpu.SemaphoreType.DMA,
        ],
        compiler_params=pltpu.CompilerParams(use_tc_tiling_on_sc=False),
    )(embedding_matrix, idx_t)
    return out.transpose(1, 0, 2)
